# baseline (speedup 1.0000x reference)
import functools

import jax
import jax.numpy as jnp
from jax import lax
from jax.experimental import pallas as pl
from jax.experimental.pallas import tpu as pltpu
from jax.experimental.pallas import tpu_sc as plsc

D = 128
S = 50
NB = 4096
SCALE_F = 10.0
NC, NS, L = 2, 16, 16
NW = NC * NS
SPAN = S * NB // NW
CH = 128
NCHUNK = SPAN // CH
NPAIR = NCHUNK // 2


def _sc_gather_scale(table, idx_flat):
    mesh = plsc.VectorSubcoreMesh(core_axis_name="c", subcore_axis_name="s")

    @functools.partial(
        pl.kernel,
        mesh=mesh,
        out_type=jax.ShapeDtypeStruct((S, NB, D), jnp.float32),
        scratch_types=[
            pltpu.VMEM((SPAN,), jnp.int32),
            pltpu.VMEM((CH, D), jnp.float32),
            pltpu.VMEM((CH, D), jnp.float32),
            pltpu.VMEM((CH, D), jnp.float32),
            pltpu.VMEM((2 * CH, D), jnp.float32),
            pltpu.VMEM((2 * CH, D), jnp.float32),
            pltpu.SemaphoreType.DMA,
            pltpu.SemaphoreType.DMA,
            pltpu.SemaphoreType.DMA,
            pltpu.SemaphoreType.DMA,
            pltpu.SemaphoreType.DMA,
        ],
    )
    def k(table_hbm, idx_hbm, out_hbm, idx_v, g0, g1, g2, p0, p1,
          gsem0, gsem1, gsem2, psem0, psem1):
        wid = lax.axis_index("s") * NC + lax.axis_index("c")
        g_base = wid * SPAN

        pltpu.sync_copy(idx_hbm.at[pl.ds(g_base, SPAN)], idx_v)

        gbufs = (g0, g1, g2)
        gsems = (gsem0, gsem1, gsem2)
        pbufs = (p0, p1)
        psems = (psem0, psem1)

        def gather(c, buf, sem):
            pltpu.async_copy(
                table_hbm.at[idx_v.at[pl.ds(c * CH, CH)]], buf, sem)

        def out_pair_slice(p):
            g = g_base + p * 2 * CH
            return out_hbm.at[g // NB, pl.ds(g % NB, 2 * CH)]

        gather(0, g0, gsem0)
        gather(1, g1, gsem1)

        NGRP = (NPAIR + 5) // 6

        def group(p0_idx, _):
            for ps in range(6):
                p = p0_idx + ps
                pbuf, psem = pbufs[ps % 2], psems[ps % 2]

                @pl.when(p < NPAIR)
                def _():
                    @pl.when(p >= 2)
                    def _():
                        pltpu.make_async_copy(
                            pbuf, out_pair_slice(p), psem).wait()

                    for h in range(2):
                        c = 2 * p + h
                        bg = (2 * ps + h) % 3
                        gbuf, gsem = gbufs[bg], gsems[bg]
                        pltpu.make_async_copy(
                            table_hbm.at[idx_v.at[pl.ds(0, CH)]], gbuf,
                            gsem).wait()

                        @pl.when(c + 2 < NCHUNK)
                        def _():
                            gather(c + 2, gbufs[(bg + 2) % 3],
                                   gsems[(bg + 2) % 3])

                        @plsc.parallel_loop(0, CH, unroll=8)
                        def _(r):
                            for j in range(D // L):
                                s = pl.ds(j * L, L)
                                pbuf[h * CH + r, s] = gbuf[r, s] * SCALE_F

                    pltpu.async_copy(pbuf, out_pair_slice(p), psem)
            return ()

        lax.fori_loop(0, NGRP, lambda i, a: group(i * 6, a), ())

        for p in range(NPAIR - 2, NPAIR):
            pltpu.make_async_copy(
                pbufs[p % 2], out_pair_slice(p), psems[p % 2]).wait()

    return k(table, idx_flat)


def kernel(x, weight):
    idx_flat = jnp.transpose(x.astype(jnp.int32)).reshape(-1)
    out_t = _sc_gather_scale(weight, idx_flat)
    return jnp.transpose(out_t, (1, 0, 2))


# speedup vs baseline: 1.0354x; 1.0354x over previous
import functools

import jax
import jax.numpy as jnp
from jax import lax
from jax.experimental import pallas as pl
from jax.experimental.pallas import tpu as pltpu
from jax.experimental.pallas import tpu_sc as plsc

D = 128
S = 50
NB = 4096
SCALE_F = 10.0
NC, NS, L = 2, 16, 16
NW = NC * NS
SPAN = S * NB // NW
CH = 128
NCHUNK = SPAN // CH


def _sc_gather_scale(table, idx_flat):
    mesh = plsc.VectorSubcoreMesh(core_axis_name="c", subcore_axis_name="s")

    @functools.partial(
        pl.kernel,
        mesh=mesh,
        out_type=jax.ShapeDtypeStruct((S, NB, D), jnp.float32),
        scratch_types=[
            pltpu.VMEM((SPAN,), jnp.int32),
            pltpu.VMEM((CH, D), jnp.float32),
            pltpu.VMEM((CH, D), jnp.float32),
            pltpu.VMEM((CH, D), jnp.float32),
            pltpu.VMEM((CH, D), jnp.float32),
            pltpu.VMEM((CH, D), jnp.float32),
            pltpu.VMEM((CH, D), jnp.float32),
            pltpu.SemaphoreType.DMA,
            pltpu.SemaphoreType.DMA,
            pltpu.SemaphoreType.DMA,
            pltpu.SemaphoreType.DMA,
            pltpu.SemaphoreType.DMA,
            pltpu.SemaphoreType.DMA,
        ],
    )
    def k(table_hbm, idx_hbm, out_hbm, idx_v, g0, g1, g2, o0, o1, o2,
          gsem0, gsem1, gsem2, osem0, osem1, osem2):
        wid = lax.axis_index("s") * NC + lax.axis_index("c")
        g_base = wid * SPAN

        pltpu.sync_copy(idx_hbm.at[pl.ds(g_base, SPAN)], idx_v)

        gbufs, obufs = (g0, g1, g2), (o0, o1, o2)
        gsems, osems = (gsem0, gsem1, gsem2), (osem0, osem1, osem2)

        def gather(c, buf, sem):
            pltpu.async_copy(
                table_hbm.at[idx_v.at[pl.ds(c * CH, CH)]], buf, sem)

        def out_slice(c):
            g = g_base + c * CH
            return out_hbm.at[g // NB, pl.ds(g % NB, CH)]

        gather(0, g0, gsem0)
        gather(1, g1, gsem1)

        NGRP = (NCHUNK + 2) // 3

        def step(c0, _):
            for b in range(3):
                c = c0 + b
                gbuf, obuf = gbufs[b], obufs[b]
                gsem, osem = gsems[b], osems[b]

                @pl.when(c < NCHUNK)
                def _():
                    pltpu.make_async_copy(
                        table_hbm.at[idx_v.at[pl.ds(0, CH)]], gbuf,
                        gsem).wait()

                    @pl.when(c + 2 < NCHUNK)
                    def _():
                        gather(c + 2, gbufs[(b + 2) % 3],
                               gsems[(b + 2) % 3])

                    @pl.when(c >= 3)
                    def _():
                        pltpu.make_async_copy(
                            obuf, out_slice(c), osem).wait()

                    @plsc.parallel_loop(0, CH, unroll=8)
                    def _(r):
                        for j in range(D // L):
                            s = pl.ds(j * L, L)
                            obuf[r, s] = gbuf[r, s] * SCALE_F

                    pltpu.async_copy(obuf, out_slice(c), osem)
            return ()

        lax.fori_loop(0, NGRP, lambda i, a: step(i * 3, a), ())

        for c in range(NCHUNK - 3, NCHUNK):
            pltpu.make_async_copy(
                obufs[c % 3], out_slice(c), osems[c % 3]).wait()

    return k(table, idx_flat)


def kernel(x, weight):
    idx_flat = jnp.transpose(x.astype(jnp.int32)).reshape(-1)
    out_t = _sc_gather_scale(weight, idx_flat)
    return jnp.transpose(out_t, (1, 0, 2))
